# baseline (speedup 1.0000x reference)
import functools

import jax
import jax.numpy as jnp
import numpy as np
from jax.experimental import pallas as pl
from jax.experimental.pallas import tpu as pltpu

_K = 1024
_D = 64
_CD = 10
_SCALE = 1.0
_ENTROPY_W = 0.1
_COMMIT_W = 0.25
_GAMMA = 1.0
_INV_TEMP = 100.0
_B, _N = 8, 4096
_T = _B * _N

_HALF = _CD // 2
_J = 1 << _HALF
_BB = 4
_TN = _N


def _one_tile(zT, wi, bi_col, wo, bo_col):
    xT = jax.lax.dot_general(
        wi, zT, (((1,), (0,)), ((), ())),
        preferred_element_type=jnp.float32) + bi_col

    pos = xT > 0.0
    q = jnp.where(pos, _SCALE, -_SCALE)

    out_t = jax.lax.dot_general(
        wo, q, (((0,), (0,)), ((), ())),
        preferred_element_type=jnp.float32) + bo_col

    d_iota = jax.lax.broadcasted_iota(jnp.int32, (_CD, 1), 0)
    bw = jnp.left_shift(1, (_CD - 1) - d_iota)
    idx_row = jnp.sum(jnp.where(pos, bw, 0), axis=0, keepdims=True)

    L = (4.0 * _INV_TEMP * _SCALE) * xT
    La = jnp.abs(L)
    t = jnp.exp(-La)
    inv1pt = 1.0 / (1.0 + t)
    sig_a = inv1pt
    sig_b = t * inv1pt
    lg1pt = jnp.log(1.0 + t)

    ent_tile = jnp.sum(lg1pt + La * sig_b)
    com_tile = jnp.sum((xT - q) ** 2)

    p = jnp.where(pos, sig_a, sig_b)
    pm = jnp.where(pos, sig_b, sig_a)

    def half(d0):
        dl = d0 + _HALF - 1
        X = jnp.concatenate([pm[dl][None, :], p[dl][None, :]], axis=0)
        for k in range(1, _HALF):
            d = dl - k
            X = jnp.concatenate([X * pm[d][None, :], X * p[d][None, :]],
                                axis=0)
        return X

    A = half(0)
    Bm = half(_HALF)
    s_delta = jax.lax.dot_general(
        A, Bm, (((1,), (1,)), ((), ())),
        preferred_element_type=jnp.float32)
    return out_t, idx_row, ent_tile, com_tile, s_delta


def _lfq_body(z_ref, wi_ref, bi_ref, wo_ref, bo_ref,
              out_ref, idx_ref, aux_ref,
              s_ref, ent_ref, com_ref):
    i = pl.program_id(0)
    nb = pl.num_programs(0)

    wi = wi_ref[...]
    wo = wo_ref[...]
    bi_col = jnp.swapaxes(bi_ref[...], 0, 1)
    bo_col = jnp.swapaxes(bo_ref[...], 0, 1)

    @pl.when(i == 0)
    def _init():
        s_ref[...] = jnp.zeros((_J, _J), jnp.float32)
        ent_ref[0, 0] = 0.0
        com_ref[0, 0] = 0.0

    for s in range(_BB):
        out_t, idx_row, ent_tile, com_tile, s_delta = _one_tile(
            z_ref[s], wi, bi_col, wo, bo_col)
        out_ref[s] = out_t
        idx_ref[pl.ds(i * _BB + s, 1), :] = idx_row
        s_ref[...] += s_delta
        ent_ref[0, 0] += ent_tile
        com_ref[0, 0] += com_tile

    @pl.when(i == nb - 1)
    def _finish():
        tn = float(_T)
        ap = s_ref[...] / tn
        cb_ent = jnp.sum(-ap * jnp.log(jnp.maximum(ap, 1e-20)))
        ps_ent = ent_ref[0, 0] / tn
        commit = com_ref[0, 0] / (tn * _CD)
        aux = (ps_ent - _GAMMA * cb_ent) * _ENTROPY_W + commit * _COMMIT_W
        aux_ref[...] = jnp.broadcast_to(aux, (1, 1))


@functools.partial(jax.jit, static_argnames=())
def kernel(z_e_x, W_in, b_in, W_out, b_out):
    grid = (_B // _BB,)
    out, idx3, aux = pl.pallas_call(
        _lfq_body,
        grid=grid,
        in_specs=[
            pl.BlockSpec((_BB, _D, _TN), lambda i: (i, 0, 0)),
            pl.BlockSpec((_CD, _D), lambda i: (0, 0)),
            pl.BlockSpec((1, _CD), lambda i: (0, 0)),
            pl.BlockSpec((_CD, _D), lambda i: (0, 0)),
            pl.BlockSpec((1, _D), lambda i: (0, 0)),
        ],
        out_specs=[
            pl.BlockSpec((_BB, _D, _TN), lambda i: (i, 0, 0)),
            pl.BlockSpec((_B, _N), lambda i: (0, 0)),
            pl.BlockSpec((1, 1), lambda i: (0, 0)),
        ],
        out_shape=[
            jax.ShapeDtypeStruct((_B, _D, _N), jnp.float32),
            jax.ShapeDtypeStruct((_B, _N), jnp.int32),
            jax.ShapeDtypeStruct((1, 1), jnp.float32),
        ],
        scratch_shapes=[
            pltpu.VMEM((_J, _J), jnp.float32),
            pltpu.SMEM((1, 1), jnp.float32),
            pltpu.SMEM((1, 1), jnp.float32),
        ],
    )(jnp.swapaxes(z_e_x, 1, 2), W_in, b_in.reshape(1, _CD),
      jnp.swapaxes(W_out, 0, 1), b_out.reshape(1, _D))
    return (jnp.swapaxes(out, 1, 2), idx3, aux.reshape(()))


# speedup vs baseline: 1.0078x; 1.0078x over previous
import functools

import jax
import jax.numpy as jnp
import numpy as np
from jax.experimental import pallas as pl
from jax.experimental.pallas import tpu as pltpu

_K = 1024
_D = 64
_CD = 10
_SCALE = 1.0
_ENTROPY_W = 0.1
_COMMIT_W = 0.25
_GAMMA = 1.0
_INV_TEMP = 100.0
_B, _N = 8, 4096
_T = _B * _N

_HALF = _CD // 2
_J = 1 << _HALF
_BB = 2
_TN = _N


def _one_tile(zT, wi, bi_col, wo, bo_col):
    xT = jax.lax.dot_general(
        wi, zT, (((1,), (0,)), ((), ())),
        preferred_element_type=jnp.float32) + bi_col

    pos = xT > 0.0
    q = jnp.where(pos, _SCALE, -_SCALE)

    out_t = jax.lax.dot_general(
        wo, q, (((0,), (0,)), ((), ())),
        preferred_element_type=jnp.float32) + bo_col

    d_iota = jax.lax.broadcasted_iota(jnp.int32, (1, _CD), 1)
    bw = jnp.left_shift(1, (_CD - 1) - d_iota).astype(jnp.float32)
    wq = jax.lax.dot_general(
        bw, q, (((1,), (0,)), ((), ())),
        preferred_element_type=jnp.float32)
    idx_row = ((wq + float(_K - 1)) * 0.5).astype(jnp.int32)

    L = (4.0 * _INV_TEMP * _SCALE) * xT
    La = jnp.abs(L)
    t = jnp.exp(-La)
    inv1pt = 1.0 / (1.0 + t)
    sig_a = inv1pt
    sig_b = t * inv1pt
    lg1pt = jnp.log(1.0 + t)

    g_tile = jnp.sum(_ENTROPY_W * (lg1pt + La * sig_b)
                     + (_COMMIT_W / _CD) * (xT * xT - 2.0 * jnp.abs(xT)))

    p = jnp.where(pos, sig_a, sig_b)
    pm = jnp.where(pos, sig_b, sig_a)

    def half(d0):
        dl = d0 + _HALF - 1
        X = jnp.concatenate([pm[dl][None, :], p[dl][None, :]], axis=0)
        for k in range(1, _HALF):
            d = dl - k
            X = jnp.concatenate([X * pm[d][None, :], X * p[d][None, :]],
                                axis=0)
        return X

    A = half(0)
    Bm = half(_HALF)
    s_delta = jax.lax.dot_general(
        A, Bm, (((1,), (1,)), ((), ())),
        preferred_element_type=jnp.float32)
    return out_t, idx_row, g_tile, s_delta


def _lfq_body(z_ref, wi_ref, bi_ref, wo_ref, bo_ref,
              out_ref, idx_ref, aux_ref,
              s_ref, g_ref):
    i = pl.program_id(0)
    nb = pl.num_programs(0)

    wi = wi_ref[...]
    wo = wo_ref[...]
    bi_col = jnp.swapaxes(bi_ref[...], 0, 1)
    bo_col = jnp.swapaxes(bo_ref[...], 0, 1)

    @pl.when(i == 0)
    def _init():
        s_ref[...] = jnp.zeros((_J, _J), jnp.float32)
        g_ref[0, 0] = 0.0

    for s in range(_BB):
        out_t, idx_row, g_tile, s_delta = _one_tile(
            z_ref[s], wi, bi_col, wo, bo_col)
        out_ref[s] = out_t
        idx_ref[pl.ds(i * _BB + s, 1), :] = idx_row
        s_ref[...] += s_delta
        g_ref[0, 0] += g_tile

    @pl.when(i == nb - 1)
    def _finish():
        tn = float(_T)
        ap = s_ref[...] / tn
        cb_ent = jnp.sum(-ap * jnp.log(jnp.maximum(ap, 1e-20)))
        aux = (g_ref[0, 0] / tn + _COMMIT_W
               - _ENTROPY_W * _GAMMA * cb_ent)
        aux_ref[...] = jnp.broadcast_to(aux, (1, 1))


@functools.partial(jax.jit, static_argnames=())
def kernel(z_e_x, W_in, b_in, W_out, b_out):
    grid = (_B // _BB,)
    out, idx3, aux = pl.pallas_call(
        _lfq_body,
        grid=grid,
        in_specs=[
            pl.BlockSpec((_BB, _D, _TN), lambda i: (i, 0, 0)),
            pl.BlockSpec((_CD, _D), lambda i: (0, 0)),
            pl.BlockSpec((1, _CD), lambda i: (0, 0)),
            pl.BlockSpec((_CD, _D), lambda i: (0, 0)),
            pl.BlockSpec((1, _D), lambda i: (0, 0)),
        ],
        out_specs=[
            pl.BlockSpec((_BB, _D, _TN), lambda i: (i, 0, 0)),
            pl.BlockSpec((_B, _N), lambda i: (0, 0)),
            pl.BlockSpec((1, 1), lambda i: (0, 0)),
        ],
        out_shape=[
            jax.ShapeDtypeStruct((_B, _D, _N), jnp.float32),
            jax.ShapeDtypeStruct((_B, _N), jnp.int32),
            jax.ShapeDtypeStruct((1, 1), jnp.float32),
        ],
        scratch_shapes=[
            pltpu.VMEM((_J, _J), jnp.float32),
            pltpu.SMEM((1, 1), jnp.float32),
        ],
    )(jnp.swapaxes(z_e_x, 1, 2), W_in, b_in.reshape(1, _CD),
      jnp.swapaxes(W_out, 0, 1), b_out.reshape(1, _D))
    return (jnp.swapaxes(out, 1, 2), idx3, aux.reshape(()))


# speedup vs baseline: 1.0088x; 1.0010x over previous
import functools

import jax
import jax.numpy as jnp
from jax.experimental import pallas as pl
from jax.experimental.pallas import tpu as pltpu

_K = 1024
_D = 64
_CD = 10
_SCALE = 1.0
_ENTROPY_W = 0.1
_COMMIT_W = 0.25
_GAMMA = 1.0
_INV_TEMP = 100.0
_B, _N = 8, 4096
_T = _B * _N

_HALF = _CD // 2
_J = 1 << _HALF
_BB = 2
_TN = _N


def _one_tile(zT, wi, bi_col, wo, bo_col):
    xT = jax.lax.dot_general(
        wi, zT, (((1,), (0,)), ((), ())),
        preferred_element_type=jnp.float32) + bi_col

    pos = xT > 0.0
    q = jnp.where(pos, _SCALE, -_SCALE)

    out_t = jax.lax.dot_general(
        wo, q, (((0,), (0,)), ((), ())),
        preferred_element_type=jnp.float32) + bo_col

    d_iota = jax.lax.broadcasted_iota(jnp.int32, (1, _CD), 1)
    bw = jnp.left_shift(1, (_CD - 1) - d_iota).astype(jnp.float32)
    wq = jax.lax.dot_general(
        bw, q, (((1,), (0,)), ((), ())),
        preferred_element_type=jnp.float32)
    idx_row = ((wq + float(_K - 1)) * 0.5).astype(jnp.int32)

    L = (4.0 * _INV_TEMP * _SCALE) * xT
    La = jnp.abs(L)
    t = jnp.exp(-La)
    inv1pt = 1.0 / (1.0 + t)
    sig_a = inv1pt
    sig_b = t * inv1pt
    lg1pt = jnp.log(1.0 + t)

    g_tile = jnp.sum(_ENTROPY_W * (lg1pt + La * sig_b)
                     + (_COMMIT_W / _CD) * (xT * xT - 2.0 * jnp.abs(xT)))

    p = jnp.where(pos, sig_a, sig_b)
    pm = jnp.where(pos, sig_b, sig_a)

    def half(d0):
        dl = d0 + _HALF - 1
        X = jnp.concatenate([pm[dl][None, :], p[dl][None, :]], axis=0)
        for k in range(1, _HALF):
            d = dl - k
            X = jnp.concatenate([X * pm[d][None, :], X * p[d][None, :]],
                                axis=0)
        return X

    A = half(0)
    Bm = half(_HALF)
    s_delta = jax.lax.dot_general(
        A, Bm, (((1,), (1,)), ((), ())),
        preferred_element_type=jnp.float32)
    return out_t, idx_row, g_tile, s_delta


def _lfq_body(z_ref, wi_ref, bi_ref, wo_ref, bo_ref,
              out_ref, idx_ref, aux_ref,
              s_ref, g_ref):
    i = pl.program_id(0)
    nb = pl.num_programs(0)

    wi = wi_ref[...]
    wo = wo_ref[...]
    bi_col = jnp.swapaxes(bi_ref[...], 0, 1)
    bo_col = jnp.swapaxes(bo_ref[...], 0, 1)

    @pl.when(i == 0)
    def _init():
        s_ref[...] = jnp.zeros((_J, _J), jnp.float32)
        g_ref[0, 0] = 0.0

    for s in range(_BB):
        out_t, idx_row, g_tile, s_delta = _one_tile(
            z_ref[s], wi, bi_col, wo, bo_col)
        out_ref[s] = out_t
        idx_ref[pl.ds(i * _BB + s, 1), :] = idx_row
        s_ref[...] += s_delta
        g_ref[0, 0] += g_tile

    @pl.when(i == nb - 1)
    def _finish():
        tn = float(_T)
        ap = s_ref[...] / tn
        cb_ent = jnp.sum(-ap * jnp.log(jnp.maximum(ap, 1e-20)))
        aux = (g_ref[0, 0] / tn + _COMMIT_W
               - _ENTROPY_W * _GAMMA * cb_ent)
        aux_ref[...] = jnp.broadcast_to(aux, (1, 1))


@functools.partial(jax.jit, static_argnames=())
def kernel(z_e_x, W_in, b_in, W_out, b_out):
    grid = (_B // _BB,)
    out, idx3, aux = pl.pallas_call(
        _lfq_body,
        grid=grid,
        in_specs=[
            pl.BlockSpec((_BB, _D, _TN), lambda i: (i, 0, 0)),
            pl.BlockSpec((_CD, _D), lambda i: (0, 0)),
            pl.BlockSpec((1, _CD), lambda i: (0, 0)),
            pl.BlockSpec((_CD, _D), lambda i: (0, 0)),
            pl.BlockSpec((1, _D), lambda i: (0, 0)),
        ],
        out_specs=[
            pl.BlockSpec((_BB, _D, _TN), lambda i: (i, 0, 0)),
            pl.BlockSpec((_B, _N), lambda i: (0, 0)),
            pl.BlockSpec((1, 1), lambda i: (0, 0)),
        ],
        out_shape=[
            jax.ShapeDtypeStruct((_B, _D, _N), jnp.float32),
            jax.ShapeDtypeStruct((_B, _N), jnp.int32),
            jax.ShapeDtypeStruct((1, 1), jnp.float32),
        ],
        scratch_shapes=[
            pltpu.VMEM((_J, _J), jnp.float32),
            pltpu.SMEM((1, 1), jnp.float32),
        ],
    )(jnp.swapaxes(z_e_x, 1, 2), W_in, b_in.reshape(1, _CD),
      jnp.swapaxes(W_out, 0, 1), b_out.reshape(1, _D))
    return (jnp.swapaxes(out, 1, 2), idx3, aux.reshape(()))
